# Initial kernel scaffold; baseline (speedup 1.0000x reference)
import functools

import jax
import jax.numpy as jnp
from jax import lax
from jax.experimental import pallas as pl
from jax.experimental.pallas import tpu as pltpu
from jax.experimental.pallas import tpu_sc as plsc

N = 10000
E = 320000
NT = 10240
EP = 323584
CH = 128
NC1 = EP // (16 * CH)
NC2 = EP // (32 * CH)
RPT = NT // 16
BLK = 1280
GRID = NT // BLK


def _sc_mesh():
    return plsc.VectorSubcoreMesh(core_axis_name="c", subcore_axis_name="s")


def _deg(dstB, zcol, ocol):
    @functools.partial(
        pl.kernel,
        out_type=jax.ShapeDtypeStruct((2 * NT, 1), jnp.float32),
        mesh=_sc_mesh(),
        scratch_types=[
            pltpu.VMEM((NC2, CH), jnp.int32),
            pltpu.VMEM((CH, 1), jnp.float32),
            pltpu.VMEM_SHARED((NT, 1), jnp.float32),
        ],
    )
    def run(dst_hbm, z_hbm, o_hbm, deg_out, idx_v, ones_v, deg_sh):
        c = lax.axis_index("c")
        s = lax.axis_index("s")
        w = c * 16 + s
        pltpu.sync_copy(dst_hbm.at[w], idx_v)
        pltpu.sync_copy(o_hbm, ones_v)
        pltpu.sync_copy(z_hbm, deg_sh.at[pl.ds(s * RPT, RPT)])
        plsc.subcore_barrier()

        @pl.loop(0, NC2)
        def _(j):
            pltpu.sync_copy(ones_v, deg_sh.at[idx_v.at[j]], add=True)

        plsc.subcore_barrier()
        pltpu.sync_copy(deg_sh.at[pl.ds(s * RPT, RPT)],
                        deg_out.at[pl.ds(c * NT + s * RPT, RPT)])

    return run(dstB, zcol, ocol)


def _agg(hs, srcX, dstX, zblk, nch):
    @functools.partial(
        pl.kernel,
        out_type=jax.ShapeDtypeStruct((2 * NT, 128), jnp.float32),
        mesh=_sc_mesh(),
        scratch_types=[
            pltpu.VMEM((nch, CH), jnp.int32),
            pltpu.VMEM((nch, CH), jnp.int32),
            pltpu.VMEM((CH, 128), jnp.float32),
            pltpu.VMEM_SHARED((NT, 128), jnp.float32),
        ],
    )
    def run(hs_hbm, src_hbm, dst_hbm, z_hbm, acc_out, srcv, dstv, buf, acc_sh):
        c = lax.axis_index("c")
        s = lax.axis_index("s")
        w = c * 16 + s
        pltpu.sync_copy(src_hbm.at[w], srcv)
        pltpu.sync_copy(dst_hbm.at[w], dstv)
        pltpu.sync_copy(z_hbm, acc_sh.at[pl.ds(s * RPT, RPT)])
        plsc.subcore_barrier()

        @pl.loop(0, nch)
        def _(j):
            pltpu.sync_copy(hs_hbm.at[srcv.at[j]], buf)
            pltpu.sync_copy(buf, acc_sh.at[dstv.at[j]], add=True)

        plsc.subcore_barrier()
        pltpu.sync_copy(acc_sh.at[pl.ds(s * RPT, RPT)],
                        acc_out.at[pl.ds(c * NT + s * RPT, RPT)])

    return run(hs, srcX, dstX, zblk)


def _mm1(xp, W1, deg2):
    deg_lo, deg_hi = deg2[:NT], deg2[NT:]

    def body(x_ref, w_ref, dlo_ref, dhi_ref, hs_ref, dinv_ref):
        i = pl.program_id(0)
        deg = 1.0 + dlo_ref[...] + dhi_ref[...]
        rows = lax.broadcasted_iota(jnp.int32, (BLK, 1), 0) + i * BLK
        dinv = jnp.where(rows < N, lax.rsqrt(deg), 0.0)
        dinv_ref[...] = dinv
        h = jnp.dot(x_ref[...], w_ref[...], preferred_element_type=jnp.float32)
        hs = h * dinv
        hs_ref[0] = hs[:, :128]
        hs_ref[1] = hs[:, 128:]

    return pl.pallas_call(
        body,
        grid=(GRID,),
        in_specs=[
            pl.BlockSpec((BLK, 128), lambda i: (i, 0)),
            pl.BlockSpec((128, 256), lambda i: (0, 0)),
            pl.BlockSpec((BLK, 1), lambda i: (i, 0)),
            pl.BlockSpec((BLK, 1), lambda i: (i, 0)),
        ],
        out_specs=[
            pl.BlockSpec((2, BLK, 128), lambda i: (0, i, 0)),
            pl.BlockSpec((BLK, 1), lambda i: (i, 0)),
        ],
        out_shape=[
            jax.ShapeDtypeStruct((2, NT, 128), jnp.float32),
            jax.ShapeDtypeStruct((NT, 1), jnp.float32),
        ],
    )(xp, W1, deg_lo, deg_hi)


def _mm2(acc1, hs1, dinv, b1r, W2):
    def body(alo, ahi, hlo, hhi, dv, b1_ref, w2_ref, out_ref):
        d = dv[...]
        y_lo = jnp.maximum((alo[0] + hlo[0]) * d + b1_ref[:, :128], 0.0)
        y_hi = jnp.maximum((ahi[0] + hhi[0]) * d + b1_ref[:, 128:], 0.0)
        h2 = (jnp.dot(y_lo, w2_ref[:128, :], preferred_element_type=jnp.float32)
              + jnp.dot(y_hi, w2_ref[128:, :], preferred_element_type=jnp.float32))
        out_ref[...] = h2 * d

    return pl.pallas_call(
        body,
        grid=(GRID,),
        in_specs=[
            pl.BlockSpec((1, BLK, 128), lambda i: (0, i, 0)),
            pl.BlockSpec((1, BLK, 128), lambda i: (1, i, 0)),
            pl.BlockSpec((1, BLK, 128), lambda i: (0, i, 0)),
            pl.BlockSpec((1, BLK, 128), lambda i: (1, i, 0)),
            pl.BlockSpec((BLK, 1), lambda i: (i, 0)),
            pl.BlockSpec((1, 256), lambda i: (0, 0)),
            pl.BlockSpec((256, 128), lambda i: (0, 0)),
        ],
        out_specs=pl.BlockSpec((BLK, 128), lambda i: (i, 0)),
        out_shape=jax.ShapeDtypeStruct((NT, 128), jnp.float32),
    )(acc1, acc1, hs1, hs1, dinv, b1r, W2)


def _mm3(acc2, hs2, dinv, b2r, Wfc, bfcr):
    def body(alo, ahi, h2, dv, b2_ref, wfc_ref, bfc_ref, out_ref):
        d = dv[...]
        y2 = jnp.maximum((alo[0] + ahi[0] + h2[...]) * d + b2_ref[...], 0.0)
        out_ref[...] = (jnp.dot(y2, wfc_ref[...], preferred_element_type=jnp.float32)
                        + bfc_ref[...])

    return pl.pallas_call(
        body,
        grid=(GRID,),
        in_specs=[
            pl.BlockSpec((1, BLK, 128), lambda i: (0, i, 0)),
            pl.BlockSpec((1, BLK, 128), lambda i: (1, i, 0)),
            pl.BlockSpec((BLK, 128), lambda i: (i, 0)),
            pl.BlockSpec((BLK, 1), lambda i: (i, 0)),
            pl.BlockSpec((1, 128), lambda i: (0, 0)),
            pl.BlockSpec((128, 1), lambda i: (0, 0)),
            pl.BlockSpec((1, 1), lambda i: (0, 0)),
        ],
        out_specs=pl.BlockSpec((BLK, 1), lambda i: (i, 0)),
        out_shape=jax.ShapeDtypeStruct((NT, 1), jnp.float32),
    )(acc2, acc2, hs2, dinv, b2r, Wfc, bfcr)


def kernel(x, edge_index, W1, b1, W2, b2, Wfc, bfc):
    xp = jnp.pad(x, ((0, NT - N), (0, 0)))
    src, dst = edge_index[0], edge_index[1]
    pad = jnp.full((EP - E,), N, dtype=jnp.int32)
    srcp = jnp.concatenate([src, pad])
    dstp = jnp.concatenate([dst, pad])
    src16 = srcp.reshape(16, NC1, CH)
    srcA = jnp.concatenate([src16, src16 + NT], axis=0)
    dst16 = dstp.reshape(16, NC1, CH)
    dstA = jnp.concatenate([dst16, dst16], axis=0)
    srcB = srcp.reshape(32, NC2, CH)
    dstB = dstp.reshape(32, NC2, CH)

    zcol = jnp.zeros((RPT, 1), jnp.float32)
    ocol = jnp.ones((CH, 1), jnp.float32)
    zblk = jnp.zeros((RPT, 128), jnp.float32)
    b1r = b1.reshape(1, 256)
    b2r = b2.reshape(1, 128)
    bfcr = bfc.reshape(1, 1)

    deg2 = _deg(dstB, zcol, ocol)
    hs1, dinv = _mm1(xp, W1, deg2)
    acc1 = _agg(hs1.reshape(2 * NT, 128), srcA, dstA, zblk, NC1)
    hs2 = _mm2(acc1.reshape(2, NT, 128), hs1, dinv, b1r, W2)
    acc2 = _agg(hs2, srcB, dstB, zblk, NC2)
    out = _mm3(acc2.reshape(2, NT, 128), hs2, dinv, b2r, Wfc, bfcr)
    return out[:N]


# speedup vs baseline: 7.4991x; 7.4991x over previous
import functools

import jax
import jax.numpy as jnp
from jax import lax
from jax.experimental import pallas as pl
from jax.experimental.pallas import tpu as pltpu
from jax.experimental.pallas import tpu_sc as plsc

N = 10000
E = 320000
NT = 10240
EP = 327680
CH = 128
NB = 16
NC1 = EP // (16 * CH)
NC2 = EP // (32 * CH)
NO1 = NC1 // NB
NO2 = NC2 // NB
RPT = NT // 16
BLK = 1280
GRID = NT // BLK


def _sc_mesh():
    return plsc.VectorSubcoreMesh(core_axis_name="c", subcore_axis_name="s")


def _deg(dstB, zcol, ocol):
    @functools.partial(
        pl.kernel,
        out_type=jax.ShapeDtypeStruct((2 * NT, 1), jnp.float32),
        mesh=_sc_mesh(),
        scratch_types=[
            pltpu.VMEM((NB, CH), jnp.int32),
            pltpu.VMEM((CH, 1), jnp.float32),
            pltpu.VMEM_SHARED((NT, 1), jnp.float32),
        ],
    )
    def run(dst_hbm, z_hbm, o_hbm, deg_out, idx_v, ones_v, deg_sh):
        c = lax.axis_index("c")
        s = lax.axis_index("s")
        w = c * 16 + s
        pltpu.sync_copy(o_hbm, ones_v)
        pltpu.sync_copy(z_hbm, deg_sh.at[pl.ds(s * RPT, RPT)])
        plsc.subcore_barrier()

        @pl.loop(0, NO2)
        def _(ob):
            pltpu.sync_copy(dst_hbm.at[w * NO2 + ob], idx_v)

            @pl.loop(0, NB)
            def _(j):
                pltpu.sync_copy(ones_v, deg_sh.at[idx_v.at[j]], add=True)

        plsc.subcore_barrier()
        pltpu.sync_copy(deg_sh.at[pl.ds(s * RPT, RPT)],
                        deg_out.at[pl.ds(c * NT + s * RPT, RPT)])

    return run(dstB, zcol, ocol)


def _agg(hs, srcX, dstX, zblk, no):
    @functools.partial(
        pl.kernel,
        out_type=jax.ShapeDtypeStruct((2 * NT, 128), jnp.float32),
        mesh=_sc_mesh(),
        scratch_types=[
            pltpu.VMEM((NB, CH), jnp.int32),
            pltpu.VMEM((NB, CH), jnp.int32),
            pltpu.VMEM((CH, 128), jnp.float32),
            pltpu.VMEM_SHARED((NT, 128), jnp.float32),
        ],
    )
    def run(hs_hbm, src_hbm, dst_hbm, z_hbm, acc_out, srcv, dstv, buf, acc_sh):
        c = lax.axis_index("c")
        s = lax.axis_index("s")
        w = c * 16 + s
        pltpu.sync_copy(z_hbm, acc_sh.at[pl.ds(s * RPT, RPT)])
        plsc.subcore_barrier()

        @pl.loop(0, no)
        def _(ob):
            pltpu.sync_copy(src_hbm.at[w * no + ob], srcv)
            pltpu.sync_copy(dst_hbm.at[w * no + ob], dstv)

            @pl.loop(0, NB)
            def _(j):
                pltpu.sync_copy(hs_hbm.at[srcv.at[j]], buf)
                pltpu.sync_copy(buf, acc_sh.at[dstv.at[j]], add=True)

        plsc.subcore_barrier()
        pltpu.sync_copy(acc_sh.at[pl.ds(s * RPT, RPT)],
                        acc_out.at[pl.ds(c * NT + s * RPT, RPT)])

    return run(hs, srcX, dstX, zblk)


def _mm1(xp, W1, deg2):
    deg_lo, deg_hi = deg2[:NT], deg2[NT:]

    def body(x_ref, w_ref, dlo_ref, dhi_ref, hs_ref, dinv_ref):
        i = pl.program_id(0)
        deg = 1.0 + dlo_ref[...] + dhi_ref[...]
        rows = lax.broadcasted_iota(jnp.int32, (BLK, 1), 0) + i * BLK
        dinv = jnp.where(rows < N, lax.rsqrt(deg), 0.0)
        dinv_ref[...] = dinv
        h = jnp.dot(x_ref[...], w_ref[...], preferred_element_type=jnp.float32)
        hs = h * dinv
        hs_ref[0] = hs[:, :128]
        hs_ref[1] = hs[:, 128:]

    return pl.pallas_call(
        body,
        grid=(GRID,),
        in_specs=[
            pl.BlockSpec((BLK, 128), lambda i: (i, 0)),
            pl.BlockSpec((128, 256), lambda i: (0, 0)),
            pl.BlockSpec((BLK, 1), lambda i: (i, 0)),
            pl.BlockSpec((BLK, 1), lambda i: (i, 0)),
        ],
        out_specs=[
            pl.BlockSpec((2, BLK, 128), lambda i: (0, i, 0)),
            pl.BlockSpec((BLK, 1), lambda i: (i, 0)),
        ],
        out_shape=[
            jax.ShapeDtypeStruct((2, NT, 128), jnp.float32),
            jax.ShapeDtypeStruct((NT, 1), jnp.float32),
        ],
    )(xp, W1, deg_lo, deg_hi)


def _mm2(acc1, hs1, dinv, b1r, W2):
    def body(alo, ahi, hlo, hhi, dv, b1_ref, w2_ref, out_ref):
        d = dv[...]
        y_lo = jnp.maximum((alo[0] + hlo[0]) * d + b1_ref[:, :128], 0.0)
        y_hi = jnp.maximum((ahi[0] + hhi[0]) * d + b1_ref[:, 128:], 0.0)
        h2 = (jnp.dot(y_lo, w2_ref[:128, :], preferred_element_type=jnp.float32)
              + jnp.dot(y_hi, w2_ref[128:, :], preferred_element_type=jnp.float32))
        out_ref[...] = h2 * d

    return pl.pallas_call(
        body,
        grid=(GRID,),
        in_specs=[
            pl.BlockSpec((1, BLK, 128), lambda i: (0, i, 0)),
            pl.BlockSpec((1, BLK, 128), lambda i: (1, i, 0)),
            pl.BlockSpec((1, BLK, 128), lambda i: (0, i, 0)),
            pl.BlockSpec((1, BLK, 128), lambda i: (1, i, 0)),
            pl.BlockSpec((BLK, 1), lambda i: (i, 0)),
            pl.BlockSpec((1, 256), lambda i: (0, 0)),
            pl.BlockSpec((256, 128), lambda i: (0, 0)),
        ],
        out_specs=pl.BlockSpec((BLK, 128), lambda i: (i, 0)),
        out_shape=jax.ShapeDtypeStruct((NT, 128), jnp.float32),
    )(acc1, acc1, hs1, hs1, dinv, b1r, W2)


def _mm3(acc2, hs2, dinv, b2r, Wfc, bfcr):
    def body(alo, ahi, h2, dv, b2_ref, wfc_ref, bfc_ref, out_ref):
        d = dv[...]
        y2 = jnp.maximum((alo[0] + ahi[0] + h2[...]) * d + b2_ref[...], 0.0)
        out_ref[...] = (jnp.dot(y2, wfc_ref[...], preferred_element_type=jnp.float32)
                        + bfc_ref[...])

    return pl.pallas_call(
        body,
        grid=(GRID,),
        in_specs=[
            pl.BlockSpec((1, BLK, 128), lambda i: (0, i, 0)),
            pl.BlockSpec((1, BLK, 128), lambda i: (1, i, 0)),
            pl.BlockSpec((BLK, 128), lambda i: (i, 0)),
            pl.BlockSpec((BLK, 1), lambda i: (i, 0)),
            pl.BlockSpec((1, 128), lambda i: (0, 0)),
            pl.BlockSpec((128, 1), lambda i: (0, 0)),
            pl.BlockSpec((1, 1), lambda i: (0, 0)),
        ],
        out_specs=pl.BlockSpec((BLK, 1), lambda i: (i, 0)),
        out_shape=jax.ShapeDtypeStruct((NT, 1), jnp.float32),
    )(acc2, acc2, hs2, dinv, b2r, Wfc, bfcr)


def kernel(x, edge_index, W1, b1, W2, b2, Wfc, bfc):
    xp = jnp.pad(x, ((0, NT - N), (0, 0)))
    src, dst = edge_index[0], edge_index[1]
    pad = jnp.full((EP - E,), N, dtype=jnp.int32)
    srcp = jnp.concatenate([src, pad])
    dstp = jnp.concatenate([dst, pad])
    src16 = srcp.reshape(16, NO1, NB, CH)
    srcA = jnp.concatenate([src16, src16 + NT], 0).reshape(32 * NO1, NB, CH)
    dst16 = dstp.reshape(16, NO1, NB, CH)
    dstA = jnp.concatenate([dst16, dst16], 0).reshape(32 * NO1, NB, CH)
    srcB = srcp.reshape(32 * NO2, NB, CH)
    dstB = dstp.reshape(32 * NO2, NB, CH)

    zcol = jnp.zeros((RPT, 1), jnp.float32)
    ocol = jnp.ones((CH, 1), jnp.float32)
    zblk = jnp.zeros((RPT, 128), jnp.float32)
    b1r = b1.reshape(1, 256)
    b2r = b2.reshape(1, 128)
    bfcr = bfc.reshape(1, 1)

    deg2 = _deg(dstB, zcol, ocol)
    hs1, dinv = _mm1(xp, W1, deg2)
    acc1 = _agg(hs1.reshape(2 * NT, 128), srcA, dstA, zblk, NO1)
    hs2 = _mm2(acc1.reshape(2, NT, 128), hs1, dinv, b1r, W2)
    acc2 = _agg(hs2, srcB, dstB, zblk, NO2)
    out = _mm3(acc2.reshape(2, NT, 128), hs2, dinv, b2r, Wfc, bfcr)
    return out[:N]


# speedup vs baseline: 8.9260x; 1.1903x over previous
import functools

import jax
import jax.numpy as jnp
from jax import lax
from jax.experimental import pallas as pl
from jax.experimental.pallas import tpu as pltpu
from jax.experimental.pallas import tpu_sc as plsc

N = 10000
E = 320000
NT = 10240
EP = 327680
CH = 128
RB = 16
EB = RB * CH
NO = EP // (32 * EB)
RPT = NT // 16
BLK = 1280
GRID = NT // BLK


def _sc_mesh():
    return plsc.VectorSubcoreMesh(core_axis_name="c", subcore_axis_name="s")


def _deg(dstB, zcol, ocol):
    @functools.partial(
        pl.kernel,
        out_type=jax.ShapeDtypeStruct((2 * NT, 1), jnp.float32),
        mesh=_sc_mesh(),
        scratch_types=[
            pltpu.VMEM((RB, CH), jnp.int32),
            pltpu.VMEM((CH, 1), jnp.float32),
            pltpu.VMEM_SHARED((NT, 1), jnp.float32),
        ],
    )
    def run(dst_hbm, z_hbm, o_hbm, deg_out, idx_v, ones_v, deg_sh):
        c = lax.axis_index("c")
        s = lax.axis_index("s")
        w = c * 16 + s
        pltpu.sync_copy(o_hbm, ones_v)
        pltpu.sync_copy(z_hbm, deg_sh.at[pl.ds(s * RPT, RPT)])
        plsc.subcore_barrier()

        @pl.loop(0, NO)
        def _(ob):
            pltpu.sync_copy(dst_hbm.at[w * NO + ob], idx_v)

            @pl.loop(0, RB)
            def _(j):
                pltpu.sync_copy(ones_v, deg_sh.at[idx_v.at[j]], add=True)

        plsc.subcore_barrier()
        pltpu.sync_copy(deg_sh.at[pl.ds(s * RPT, RPT)],
                        deg_out.at[pl.ds(c * NT + s * RPT, RPT)])

    return run(dstB, zcol, ocol)


def _agg(tab, srcB, dstB, zblk):
    @functools.partial(
        pl.kernel,
        out_type=jax.ShapeDtypeStruct((2 * NT, 128), jnp.float32),
        mesh=_sc_mesh(),
        scratch_types=[
            pltpu.VMEM((RB, CH), jnp.int32),
            pltpu.VMEM((RB, CH), jnp.int32),
            pltpu.VMEM((CH, 128), jnp.float32),
            pltpu.VMEM_SHARED((NT, 128), jnp.float32),
        ],
    )
    def run(tab_hbm, src_hbm, dst_hbm, z_hbm, acc_out, srcv, dstv, buf, acc_sh):
        c = lax.axis_index("c")
        s = lax.axis_index("s")
        w = c * 16 + s
        pltpu.sync_copy(z_hbm, acc_sh.at[pl.ds(s * RPT, RPT)])
        plsc.subcore_barrier()

        @pl.loop(0, NO)
        def _(ob):
            pltpu.sync_copy(src_hbm.at[w * NO + ob], srcv)
            pltpu.sync_copy(dst_hbm.at[w * NO + ob], dstv)

            @pl.loop(0, RB)
            def _(j):
                pltpu.sync_copy(tab_hbm.at[srcv.at[j]], buf)
                pltpu.sync_copy(buf, acc_sh.at[dstv.at[j]], add=True)

        plsc.subcore_barrier()
        pltpu.sync_copy(acc_sh.at[pl.ds(s * RPT, RPT)],
                        acc_out.at[pl.ds(c * NT + s * RPT, RPT)])

    return run(tab, srcB, dstB, zblk)


def _pre(xp, deg2):
    deg_lo, deg_hi = deg2[:NT], deg2[NT:]

    def body(x_ref, dlo_ref, dhi_ref, xs_ref, dinv_ref):
        i = pl.program_id(0)
        deg = 1.0 + dlo_ref[...] + dhi_ref[...]
        rows = lax.broadcasted_iota(jnp.int32, (BLK, 1), 0) + i * BLK
        dinv = jnp.where(rows < N, lax.rsqrt(deg), 0.0)
        dinv_ref[...] = dinv
        xs_ref[...] = x_ref[...] * dinv

    return pl.pallas_call(
        body,
        grid=(GRID,),
        in_specs=[
            pl.BlockSpec((BLK, 128), lambda i: (i, 0)),
            pl.BlockSpec((BLK, 1), lambda i: (i, 0)),
            pl.BlockSpec((BLK, 1), lambda i: (i, 0)),
        ],
        out_specs=[
            pl.BlockSpec((BLK, 128), lambda i: (i, 0)),
            pl.BlockSpec((BLK, 1), lambda i: (i, 0)),
        ],
        out_shape=[
            jax.ShapeDtypeStruct((NT, 128), jnp.float32),
            jax.ShapeDtypeStruct((NT, 1), jnp.float32),
        ],
    )(xp, deg_lo, deg_hi)


def _mmA(accx, xs, dinv, W1, b1r, W2):
    def body(alo, ahi, xs_ref, dv, w1_ref, b1_ref, w2_ref, out_ref):
        d = dv[...]
        xagg = (alo[0] + ahi[0] + xs_ref[...]) * d
        y1 = jnp.maximum(
            jnp.dot(xagg, w1_ref[...], preferred_element_type=jnp.float32)
            + b1_ref[...], 0.0)
        h2 = jnp.dot(y1, w2_ref[...], preferred_element_type=jnp.float32)
        out_ref[...] = h2 * d

    return pl.pallas_call(
        body,
        grid=(GRID,),
        in_specs=[
            pl.BlockSpec((1, BLK, 128), lambda i: (0, i, 0)),
            pl.BlockSpec((1, BLK, 128), lambda i: (1, i, 0)),
            pl.BlockSpec((BLK, 128), lambda i: (i, 0)),
            pl.BlockSpec((BLK, 1), lambda i: (i, 0)),
            pl.BlockSpec((128, 256), lambda i: (0, 0)),
            pl.BlockSpec((1, 256), lambda i: (0, 0)),
            pl.BlockSpec((256, 128), lambda i: (0, 0)),
        ],
        out_specs=pl.BlockSpec((BLK, 128), lambda i: (i, 0)),
        out_shape=jax.ShapeDtypeStruct((NT, 128), jnp.float32),
    )(accx, accx, xs, dinv, W1, b1r, W2)


def _mm3(acc2, hs2, dinv, b2r, Wfc, bfcr):
    def body(alo, ahi, h2, dv, b2_ref, wfc_ref, bfc_ref, out_ref):
        d = dv[...]
        y2 = jnp.maximum((alo[0] + ahi[0] + h2[...]) * d + b2_ref[...], 0.0)
        out_ref[...] = (jnp.dot(y2, wfc_ref[...], preferred_element_type=jnp.float32)
                        + bfc_ref[...])

    return pl.pallas_call(
        body,
        grid=(GRID,),
        in_specs=[
            pl.BlockSpec((1, BLK, 128), lambda i: (0, i, 0)),
            pl.BlockSpec((1, BLK, 128), lambda i: (1, i, 0)),
            pl.BlockSpec((BLK, 128), lambda i: (i, 0)),
            pl.BlockSpec((BLK, 1), lambda i: (i, 0)),
            pl.BlockSpec((1, 128), lambda i: (0, 0)),
            pl.BlockSpec((128, 1), lambda i: (0, 0)),
            pl.BlockSpec((1, 1), lambda i: (0, 0)),
        ],
        out_specs=pl.BlockSpec((BLK, 1), lambda i: (i, 0)),
        out_shape=jax.ShapeDtypeStruct((NT, 1), jnp.float32),
    )(acc2, acc2, hs2, dinv, b2r, Wfc, bfcr)


def kernel(x, edge_index, W1, b1, W2, b2, Wfc, bfc):
    xp = jnp.pad(x, ((0, NT - N), (0, 0)))
    src, dst = edge_index[0], edge_index[1]
    pad = jnp.full((EP - E,), N, dtype=jnp.int32)
    srcp = jnp.concatenate([src, pad])
    dstp = jnp.concatenate([dst, pad])
    srcB = srcp.reshape(32 * NO, RB, CH)
    dstB = dstp.reshape(32 * NO, RB, CH)

    zcol = jnp.zeros((RPT, 1), jnp.float32)
    ocol = jnp.ones((CH, 1), jnp.float32)
    zblk = jnp.zeros((RPT, 128), jnp.float32)
    b1r = b1.reshape(1, 256)
    b2r = b2.reshape(1, 128)
    bfcr = bfc.reshape(1, 1)

    deg2 = _deg(dstB, zcol, ocol)
    xs, dinv = _pre(xp, deg2)
    accx = _agg(xs, srcB, dstB, zblk)
    hs2 = _mmA(accx.reshape(2, NT, 128), xs, dinv, W1, b1r, W2)
    acc2 = _agg(hs2, srcB, dstB, zblk)
    out = _mm3(acc2.reshape(2, NT, 128), hs2, dinv, b2r, Wfc, bfcr)
    return out[:N]


# speedup vs baseline: 9.5674x; 1.0718x over previous
import dataclasses
import functools

import jax
import jax.numpy as jnp
from jax import lax
from jax.experimental import pallas as pl
from jax.experimental.pallas import tpu as pltpu
from jax.experimental.pallas import tpu_sc as plsc

N = 10000
E = 320000
NT = 10240
EP = 327680
CH = 128
RB = 16
EB = RB * CH
NO = EP // (32 * EB)
RPT = NT // 16
BLK = 1280
GRID = NT // BLK


def _sc_mesh():
    return plsc.VectorSubcoreMesh(core_axis_name="c", subcore_axis_name="s")


def _sc_params():
    cp = pltpu.CompilerParams()
    if "needs_layout_passes" in pltpu.CompilerParams.__dataclass_fields__:
        cp = dataclasses.replace(cp, needs_layout_passes=False)
    return cp


def _deg(dstB):
    @functools.partial(
        pl.kernel,
        out_type=jax.ShapeDtypeStruct((2 * NT, 16), jnp.float32),
        mesh=_sc_mesh(),
        compiler_params=_sc_params(),
        scratch_types=[
            pltpu.VMEM((RB, CH), jnp.int32),
            pltpu.VMEM((NT,), jnp.float32),
            pltpu.VMEM((16, RPT), jnp.float32),
            pltpu.VMEM((RPT, 16), jnp.float32),
            pltpu.VMEM_SHARED((16 * NT,), jnp.float32),
        ],
    )
    def run(dst_hbm, deg_out, idx_v, deg_l, buf16, res2, share):
        c = lax.axis_index("c")
        s = lax.axis_index("s")
        w = c * 16 + s

        @pl.loop(0, NT // 16)
        def _(i):
            deg_l[pl.ds(i * 16, 16)] = jnp.zeros((16,), jnp.float32)

        ones16 = jnp.ones((16,), jnp.float32)

        @pl.loop(0, NO)
        def _(ob):
            pltpu.sync_copy(dst_hbm.at[w * NO + ob], idx_v)

            @pl.loop(0, RB)
            def _(j):
                @pl.loop(0, CH // 16)
                def _(m):
                    idx = idx_v[j, pl.ds(m * 16, 16)]
                    plsc.addupdate_scatter(deg_l, [idx], ones16)

        pltpu.sync_copy(deg_l, share.at[pl.ds(s * NT, NT)])
        plsc.subcore_barrier()

        @pl.loop(0, 16)
        def _(t):
            pltpu.sync_copy(share.at[pl.ds(t * NT + s * RPT, RPT)], buf16.at[t])

        zer16 = jnp.zeros((16,), jnp.int32)
        lane = lax.iota(jnp.int32, 16)

        @pl.loop(0, RPT // 16)
        def _(i):
            v = jnp.zeros((16,), jnp.float32)
            for t in range(16):
                v = v + buf16[t, pl.ds(i * 16, 16)]
            plsc.store_scatter(res2, [i * 16 + lane, zer16], v)

        pltpu.sync_copy(res2, deg_out.at[pl.ds(c * NT + s * RPT, RPT)])

    return run(dstB)


def _agg(tab, srcB, dstB, zblk):
    @functools.partial(
        pl.kernel,
        out_type=jax.ShapeDtypeStruct((2 * NT, 128), jnp.float32),
        mesh=_sc_mesh(),
        scratch_types=[
            pltpu.VMEM((RB, CH), jnp.int32),
            pltpu.VMEM((RB, CH), jnp.int32),
            pltpu.VMEM((CH, 128), jnp.float32),
            pltpu.VMEM_SHARED((NT, 128), jnp.float32),
        ],
    )
    def run(tab_hbm, src_hbm, dst_hbm, z_hbm, acc_out, srcv, dstv, buf, acc_sh):
        c = lax.axis_index("c")
        s = lax.axis_index("s")
        w = c * 16 + s
        pltpu.sync_copy(z_hbm, acc_sh.at[pl.ds(s * RPT, RPT)])
        plsc.subcore_barrier()

        @pl.loop(0, NO)
        def _(ob):
            pltpu.sync_copy(src_hbm.at[w * NO + ob], srcv)
            pltpu.sync_copy(dst_hbm.at[w * NO + ob], dstv)

            @pl.loop(0, RB)
            def _(j):
                pltpu.sync_copy(tab_hbm.at[srcv.at[j]], buf)
                pltpu.sync_copy(buf, acc_sh.at[dstv.at[j]], add=True)

        plsc.subcore_barrier()
        pltpu.sync_copy(acc_sh.at[pl.ds(s * RPT, RPT)],
                        acc_out.at[pl.ds(c * NT + s * RPT, RPT)])

    return run(tab, srcB, dstB, zblk)


def _pre(xp, deg2):
    deg_lo, deg_hi = deg2[:NT], deg2[NT:]

    def body(x_ref, dlo_ref, dhi_ref, xs_ref, dinv_ref):
        i = pl.program_id(0)
        deg = 1.0 + dlo_ref[:, :1] + dhi_ref[:, :1]
        rows = lax.broadcasted_iota(jnp.int32, (BLK, 1), 0) + i * BLK
        dinv = jnp.where(rows < N, lax.rsqrt(deg), 0.0)
        dinv_ref[...] = dinv
        xs_ref[...] = x_ref[...] * dinv

    return pl.pallas_call(
        body,
        grid=(GRID,),
        in_specs=[
            pl.BlockSpec((BLK, 128), lambda i: (i, 0)),
            pl.BlockSpec((BLK, 16), lambda i: (i, 0)),
            pl.BlockSpec((BLK, 16), lambda i: (i, 0)),
        ],
        out_specs=[
            pl.BlockSpec((BLK, 128), lambda i: (i, 0)),
            pl.BlockSpec((BLK, 1), lambda i: (i, 0)),
        ],
        out_shape=[
            jax.ShapeDtypeStruct((NT, 128), jnp.float32),
            jax.ShapeDtypeStruct((NT, 1), jnp.float32),
        ],
    )(xp, deg_lo, deg_hi)


def _mmA(accx, xs, dinv, W1, b1r, W2):
    def body(alo, ahi, xs_ref, dv, w1_ref, b1_ref, w2_ref, out_ref):
        d = dv[...]
        xagg = (alo[0] + ahi[0] + xs_ref[...]) * d
        y1 = jnp.maximum(
            jnp.dot(xagg, w1_ref[...], preferred_element_type=jnp.float32)
            + b1_ref[...], 0.0)
        h2 = jnp.dot(y1, w2_ref[...], preferred_element_type=jnp.float32)
        out_ref[...] = h2 * d

    return pl.pallas_call(
        body,
        grid=(GRID,),
        in_specs=[
            pl.BlockSpec((1, BLK, 128), lambda i: (0, i, 0)),
            pl.BlockSpec((1, BLK, 128), lambda i: (1, i, 0)),
            pl.BlockSpec((BLK, 128), lambda i: (i, 0)),
            pl.BlockSpec((BLK, 1), lambda i: (i, 0)),
            pl.BlockSpec((128, 256), lambda i: (0, 0)),
            pl.BlockSpec((1, 256), lambda i: (0, 0)),
            pl.BlockSpec((256, 128), lambda i: (0, 0)),
        ],
        out_specs=pl.BlockSpec((BLK, 128), lambda i: (i, 0)),
        out_shape=jax.ShapeDtypeStruct((NT, 128), jnp.float32),
    )(accx, accx, xs, dinv, W1, b1r, W2)


def _mm3(acc2, hs2, dinv, b2r, Wfc, bfcr):
    def body(alo, ahi, h2, dv, b2_ref, wfc_ref, bfc_ref, out_ref):
        d = dv[...]
        y2 = jnp.maximum((alo[0] + ahi[0] + h2[...]) * d + b2_ref[...], 0.0)
        out_ref[...] = (jnp.dot(y2, wfc_ref[...], preferred_element_type=jnp.float32)
                        + bfc_ref[...])

    return pl.pallas_call(
        body,
        grid=(GRID,),
        in_specs=[
            pl.BlockSpec((1, BLK, 128), lambda i: (0, i, 0)),
            pl.BlockSpec((1, BLK, 128), lambda i: (1, i, 0)),
            pl.BlockSpec((BLK, 128), lambda i: (i, 0)),
            pl.BlockSpec((BLK, 1), lambda i: (i, 0)),
            pl.BlockSpec((1, 128), lambda i: (0, 0)),
            pl.BlockSpec((128, 1), lambda i: (0, 0)),
            pl.BlockSpec((1, 1), lambda i: (0, 0)),
        ],
        out_specs=pl.BlockSpec((BLK, 1), lambda i: (i, 0)),
        out_shape=jax.ShapeDtypeStruct((NT, 1), jnp.float32),
    )(acc2, acc2, hs2, dinv, b2r, Wfc, bfcr)


def kernel(x, edge_index, W1, b1, W2, b2, Wfc, bfc):
    xp = jnp.pad(x, ((0, NT - N), (0, 0)))
    src, dst = edge_index[0], edge_index[1]
    pad = jnp.full((EP - E,), N, dtype=jnp.int32)
    srcp = jnp.concatenate([src, pad])
    dstp = jnp.concatenate([dst, pad])
    srcB = srcp.reshape(32 * NO, RB, CH)
    dstB = dstp.reshape(32 * NO, RB, CH)

    zblk = jnp.zeros((RPT, 128), jnp.float32)
    b1r = b1.reshape(1, 256)
    b2r = b2.reshape(1, 128)
    bfcr = bfc.reshape(1, 1)

    deg2 = _deg(dstB)
    xs, dinv = _pre(xp, deg2)
    accx = _agg(xs, srcB, dstB, zblk)
    hs2 = _mmA(accx.reshape(2, NT, 128), xs, dinv, W1, b1r, W2)
    acc2 = _agg(hs2, srcB, dstB, zblk)
    out = _mm3(acc2.reshape(2, NT, 128), hs2, dinv, b2r, Wfc, bfcr)
    return out[:N]


# speedup vs baseline: 9.6448x; 1.0081x over previous
import dataclasses
import functools

import jax
import jax.numpy as jnp
from jax import lax
from jax.experimental import pallas as pl
from jax.experimental.pallas import tpu as pltpu
from jax.experimental.pallas import tpu_sc as plsc

N = 10000
E = 320000
NT = 10240
EP = 327680
CH = 128
RB = 16
EB = RB * CH
NO = EP // (32 * EB)
RPT = NT // 16
BLK = 1280
GRID = NT // BLK


def _sc_mesh():
    return plsc.VectorSubcoreMesh(core_axis_name="c", subcore_axis_name="s")


def _sc_params():
    cp = pltpu.CompilerParams()
    if "needs_layout_passes" in pltpu.CompilerParams.__dataclass_fields__:
        cp = dataclasses.replace(cp, needs_layout_passes=False)
    return cp


def _deg(dstB):
    @functools.partial(
        pl.kernel,
        out_type=jax.ShapeDtypeStruct((2 * NT, 16), jnp.float32),
        mesh=_sc_mesh(),
        compiler_params=_sc_params(),
        scratch_types=[
            pltpu.VMEM((RB, CH), jnp.int32),
            pltpu.VMEM((NT,), jnp.float32),
            pltpu.VMEM((16, RPT), jnp.float32),
            pltpu.VMEM((RPT, 16), jnp.float32),
            pltpu.VMEM_SHARED((16 * NT,), jnp.float32),
        ],
    )
    def run(dst_hbm, deg_out, idx_v, deg_l, buf16, res2, share):
        c = lax.axis_index("c")
        s = lax.axis_index("s")
        w = c * 16 + s

        @pl.loop(0, NT // 16)
        def _(i):
            deg_l[pl.ds(i * 16, 16)] = jnp.zeros((16,), jnp.float32)

        ones16 = jnp.ones((16,), jnp.float32)

        @pl.loop(0, NO)
        def _(ob):
            pltpu.sync_copy(dst_hbm.at[w * NO + ob], idx_v)

            @pl.loop(0, RB)
            def _(j):
                @pl.loop(0, CH // 16)
                def _(m):
                    idx = idx_v[j, pl.ds(m * 16, 16)]
                    plsc.addupdate_scatter(deg_l, [idx], ones16)

        pltpu.sync_copy(deg_l, share.at[pl.ds(s * NT, NT)])
        plsc.subcore_barrier()

        @pl.loop(0, 16)
        def _(t):
            pltpu.sync_copy(share.at[pl.ds(t * NT + s * RPT, RPT)], buf16.at[t])

        zer16 = jnp.zeros((16,), jnp.int32)
        lane = lax.iota(jnp.int32, 16)

        @pl.loop(0, RPT // 16)
        def _(i):
            v = jnp.zeros((16,), jnp.float32)
            for t in range(16):
                v = v + buf16[t, pl.ds(i * 16, 16)]
            plsc.store_scatter(res2, [i * 16 + lane, zer16], v)

        pltpu.sync_copy(res2, deg_out.at[pl.ds(c * NT + s * RPT, RPT)])

    return run(dstB)


def _agg(tab, srcB, dstB, zblk):
    @functools.partial(
        pl.kernel,
        out_type=jax.ShapeDtypeStruct((2 * NT, 128), jnp.float32),
        mesh=_sc_mesh(),
        scratch_types=[
            pltpu.VMEM((RB, CH), jnp.int32),
            pltpu.VMEM((RB, CH), jnp.int32),
            pltpu.VMEM((RB, CH), jnp.int32),
            pltpu.VMEM((RB, CH), jnp.int32),
            pltpu.VMEM((CH, 128), jnp.float32),
            pltpu.VMEM_SHARED((NT, 128), jnp.float32),
            pltpu.SemaphoreType.DMA,
        ],
    )
    def run(tab_hbm, src_hbm, dst_hbm, z_hbm, acc_out,
            srcv0, srcv1, dstv0, dstv1, buf, acc_sh, isem):
        c = lax.axis_index("c")
        s = lax.axis_index("s")
        w = c * 16 + s
        pltpu.sync_copy(z_hbm, acc_sh.at[pl.ds(s * RPT, RPT)])
        srcs, dsts = (srcv0, srcv1), (dstv0, dstv1)
        pend = (pltpu.async_copy(src_hbm.at[w * NO], srcv0, isem),
                pltpu.async_copy(dst_hbm.at[w * NO], dstv0, isem))
        plsc.subcore_barrier()

        for ob in range(NO):
            sv, dv = srcs[ob % 2], dsts[ob % 2]
            pend[0].wait()
            pend[1].wait()
            if ob + 1 < NO:
                pend = (pltpu.async_copy(src_hbm.at[w * NO + ob + 1],
                                         srcs[(ob + 1) % 2], isem),
                        pltpu.async_copy(dst_hbm.at[w * NO + ob + 1],
                                         dsts[(ob + 1) % 2], isem))

            @pl.loop(0, RB)
            def _(j, sv=sv, dv=dv):
                pltpu.sync_copy(tab_hbm.at[sv.at[j]], buf)
                pltpu.sync_copy(buf, acc_sh.at[dv.at[j]], add=True)

        plsc.subcore_barrier()
        pltpu.sync_copy(acc_sh.at[pl.ds(s * RPT, RPT)],
                        acc_out.at[pl.ds(c * NT + s * RPT, RPT)])

    return run(tab, srcB, dstB, zblk)


def _pre(xp, deg2):
    deg_lo, deg_hi = deg2[:NT], deg2[NT:]

    def body(x_ref, dlo_ref, dhi_ref, xs_ref, dinv_ref):
        i = pl.program_id(0)
        deg = 1.0 + dlo_ref[:, :1] + dhi_ref[:, :1]
        rows = lax.broadcasted_iota(jnp.int32, (BLK, 1), 0) + i * BLK
        dinv = jnp.where(rows < N, lax.rsqrt(deg), 0.0)
        dinv_ref[...] = dinv
        xs_ref[...] = x_ref[...] * dinv

    return pl.pallas_call(
        body,
        grid=(GRID,),
        in_specs=[
            pl.BlockSpec((BLK, 128), lambda i: (i, 0)),
            pl.BlockSpec((BLK, 16), lambda i: (i, 0)),
            pl.BlockSpec((BLK, 16), lambda i: (i, 0)),
        ],
        out_specs=[
            pl.BlockSpec((BLK, 128), lambda i: (i, 0)),
            pl.BlockSpec((BLK, 1), lambda i: (i, 0)),
        ],
        out_shape=[
            jax.ShapeDtypeStruct((NT, 128), jnp.float32),
            jax.ShapeDtypeStruct((NT, 1), jnp.float32),
        ],
    )(xp, deg_lo, deg_hi)


def _mmA(accx, xs, dinv, W1, b1r, W2):
    def body(alo, ahi, xs_ref, dv, w1_ref, b1_ref, w2_ref, out_ref):
        d = dv[...]
        xagg = (alo[0] + ahi[0] + xs_ref[...]) * d
        y1 = jnp.maximum(
            jnp.dot(xagg, w1_ref[...], preferred_element_type=jnp.float32)
            + b1_ref[...], 0.0)
        h2 = jnp.dot(y1, w2_ref[...], preferred_element_type=jnp.float32)
        out_ref[...] = h2 * d

    return pl.pallas_call(
        body,
        grid=(GRID,),
        in_specs=[
            pl.BlockSpec((1, BLK, 128), lambda i: (0, i, 0)),
            pl.BlockSpec((1, BLK, 128), lambda i: (1, i, 0)),
            pl.BlockSpec((BLK, 128), lambda i: (i, 0)),
            pl.BlockSpec((BLK, 1), lambda i: (i, 0)),
            pl.BlockSpec((128, 256), lambda i: (0, 0)),
            pl.BlockSpec((1, 256), lambda i: (0, 0)),
            pl.BlockSpec((256, 128), lambda i: (0, 0)),
        ],
        out_specs=pl.BlockSpec((BLK, 128), lambda i: (i, 0)),
        out_shape=jax.ShapeDtypeStruct((NT, 128), jnp.float32),
    )(accx, accx, xs, dinv, W1, b1r, W2)


def _mm3(acc2, hs2, dinv, b2r, Wfc, bfcr):
    def body(alo, ahi, h2, dv, b2_ref, wfc_ref, bfc_ref, out_ref):
        d = dv[...]
        y2 = jnp.maximum((alo[0] + ahi[0] + h2[...]) * d + b2_ref[...], 0.0)
        out_ref[...] = (jnp.dot(y2, wfc_ref[...], preferred_element_type=jnp.float32)
                        + bfc_ref[...])

    return pl.pallas_call(
        body,
        grid=(GRID,),
        in_specs=[
            pl.BlockSpec((1, BLK, 128), lambda i: (0, i, 0)),
            pl.BlockSpec((1, BLK, 128), lambda i: (1, i, 0)),
            pl.BlockSpec((BLK, 128), lambda i: (i, 0)),
            pl.BlockSpec((BLK, 1), lambda i: (i, 0)),
            pl.BlockSpec((1, 128), lambda i: (0, 0)),
            pl.BlockSpec((128, 1), lambda i: (0, 0)),
            pl.BlockSpec((1, 1), lambda i: (0, 0)),
        ],
        out_specs=pl.BlockSpec((BLK, 1), lambda i: (i, 0)),
        out_shape=jax.ShapeDtypeStruct((NT, 1), jnp.float32),
    )(acc2, acc2, hs2, dinv, b2r, Wfc, bfcr)


def kernel(x, edge_index, W1, b1, W2, b2, Wfc, bfc):
    xp = jnp.pad(x, ((0, NT - N), (0, 0)))
    src, dst = edge_index[0], edge_index[1]
    pad = jnp.full((EP - E,), N, dtype=jnp.int32)
    srcp = jnp.concatenate([src, pad])
    dstp = jnp.concatenate([dst, pad])
    srcB = srcp.reshape(32 * NO, RB, CH)
    dstB = dstp.reshape(32 * NO, RB, CH)

    zblk = jnp.zeros((RPT, 128), jnp.float32)
    b1r = b1.reshape(1, 256)
    b2r = b2.reshape(1, 128)
    bfcr = bfc.reshape(1, 1)

    deg2 = _deg(dstB)
    xs, dinv = _pre(xp, deg2)
    accx = _agg(xs, srcB, dstB, zblk)
    hs2 = _mmA(accx.reshape(2, NT, 128), xs, dinv, W1, b1r, W2)
    acc2 = _agg(hs2, srcB, dstB, zblk)
    out = _mm3(acc2.reshape(2, NT, 128), hs2, dinv, b2r, Wfc, bfcr)
    return out[:N]
